# Initial kernel scaffold; baseline (speedup 1.0000x reference)
import jax
import jax.numpy as jnp
from jax.experimental import pallas as pl


def kernel(x, edge_index, edge_features, W1, b1, W2, b2, mW1, mb1, mW2, mb2):
    raise NotImplementedError("write your pallas kernel here")


# speedup vs baseline: 3.4625x; 3.4625x over previous
import functools

import jax
import jax.numpy as jnp
from jax import lax
from jax.experimental import pallas as pl
from jax.experimental.pallas import tpu as pltpu
from jax.experimental.pallas import tpu_sc as plsc

NC, NS, L = 2, 16, 16
NW = NC * NS
CHUNK = 128
CPW = 80
EPW = CPW * CHUNK
E_PAD = NW * EPW
N_PAD = 10240
RPS = N_PAD // NS

_MESH = plsc.VectorSubcoreMesh(
    core_axis_name="c", subcore_axis_name="s", num_cores=NC, num_subcores=NS)


def _wid(c, s):
    return c * NS + s


_HR = N_PAD // 128


def _deg_body(dst2d, zeros_hbm, id_hbm, out_hbm, idx_v, id_v, hist_v, acc_sh,
              sem):
    del sem
    c = lax.axis_index("c")
    s = lax.axis_index("s")
    w = _wid(c, s)
    pltpu.sync_copy(dst2d.at[pl.ds(w * CPW, CPW)], idx_v)
    pltpu.sync_copy(id_hbm, id_v)

    @pl.when(s < _HR // 8)
    def _():
        pltpu.sync_copy(zeros_hbm, acc_sh.at[pl.ds(pl.multiple_of(s * 8, 8), 8)])
    zeros = jnp.zeros((L,), jnp.float32)
    ones = jnp.ones((L,), jnp.float32)

    def zb(f, _):
        hist_v[f // 8, pl.ds(pl.multiple_of((f % 8) * L, L), L)] = zeros
        return 0

    lax.fori_loop(0, _HR * 8, zb, 0)

    def body(f, _):
        iv = idx_v[f // 8, pl.ds(pl.multiple_of((f % 8) * L, L), L)]
        rows = lax.shift_right_logical(iv, 7)
        cols = jnp.bitwise_and(iv, 127)
        plsc.addupdate_scatter(hist_v, [rows, cols], ones)
        return 0

    lax.fori_loop(0, CPW * 8, body, 0)
    plsc.subcore_barrier()
    pltpu.sync_copy(hist_v, acc_sh.at[id_v], add=True)
    plsc.subcore_barrier()

    @pl.when(s < _HR // 8)
    def _():
        pltpu.sync_copy(acc_sh.at[pl.ds(pl.multiple_of(s * 8, 8), 8)],
                        out_hbm.at[c, pl.ds(pl.multiple_of(s * 8, 8), 8)])


_deg_call = pl.kernel(
    _deg_body,
    out_type=jax.ShapeDtypeStruct((NC, _HR, 128), jnp.float32),
    mesh=_MESH,
    compiler_params=pltpu.CompilerParams(needs_layout_passes=False),
    scratch_types=[
        pltpu.VMEM((CPW, CHUNK), jnp.int32),
        pltpu.VMEM((_HR,), jnp.int32),
        pltpu.VMEM((_HR, 128), jnp.float32),
        pltpu.VMEM_SHARED((_HR, 128), jnp.float32),
        pltpu.SemaphoreType.DMA,
    ],
)


def _segsum_body(table, src2d, dst2d, zeros_hbm, out_hbm,
                 sidx_v, didx_v, rows_v, acc_sh, sem):
    c = lax.axis_index("c")
    s = lax.axis_index("s")
    w = _wid(c, s)
    pltpu.sync_copy(src2d.at[pl.ds(w * CPW, CPW)], sidx_v)
    pltpu.sync_copy(dst2d.at[pl.ds(w * CPW, CPW)], didx_v)
    pltpu.sync_copy(zeros_hbm, acc_sh.at[pl.ds(s * RPS, RPS)])
    plsc.subcore_barrier()

    def body(j, _):
        pltpu.async_copy(table.at[sidx_v.at[j]], rows_v, sem).wait()
        pltpu.sync_copy(rows_v, acc_sh.at[didx_v.at[j]], add=True)
        return 0

    lax.fori_loop(0, CPW, body, 0)
    plsc.subcore_barrier()
    pltpu.sync_copy(acc_sh.at[pl.ds(s * RPS, RPS)],
                    out_hbm.at[c, pl.ds(s * RPS, RPS)])


_segsum_call = pl.kernel(
    _segsum_body,
    out_type=jax.ShapeDtypeStruct((NC, N_PAD, 128), jnp.float32),
    mesh=_MESH,
    compiler_params=pltpu.CompilerParams(needs_layout_passes=False),
    scratch_types=[
        pltpu.VMEM((CPW, CHUNK), jnp.int32),
        pltpu.VMEM((CPW, CHUNK), jnp.int32),
        pltpu.VMEM((CHUNK, 128), jnp.float32),
        pltpu.VMEM_SHARED((N_PAD, 128), jnp.float32),
        pltpu.SemaphoreType.DMA,
    ],
)


def _edge_body(a_hbm, b_hbm, src2d, dst2d, ef0_hbm, ef1_hbm, consts_hbm,
               out_hbm, sidx_v, didx_v, ef0_v, ef1_v, consts_v,
               arows_v, brows_v, out_v, sema, semb):
    c = lax.axis_index("c")
    s = lax.axis_index("s")
    w = _wid(c, s)
    pltpu.sync_copy(src2d.at[pl.ds(w * CPW, CPW)], sidx_v)
    pltpu.sync_copy(dst2d.at[pl.ds(w * CPW, CPW)], didx_v)
    pltpu.sync_copy(ef0_hbm.at[pl.ds(w * EPW, EPW)], ef0_v)
    pltpu.sync_copy(ef1_hbm.at[pl.ds(w * EPW, EPW)], ef1_v)
    pltpu.sync_copy(consts_hbm, consts_v)
    iota = lax.iota(jnp.int32, L)
    r0 = [consts_v[0, pl.ds(k * L, L)] for k in range(128 // L)]
    r1 = [consts_v[1, pl.ds(k * L, L)] for k in range(128 // L)]
    wv = [consts_v[2, pl.ds(k * L, L)] for k in range(128 // L)]
    mb2v = consts_v[3, pl.ds(0, L)]

    def body(j, _):
        pltpu.async_copy(a_hbm.at[sidx_v.at[j]], arows_v, sema).wait()
        pltpu.async_copy(b_hbm.at[didx_v.at[j]], brows_v, semb).wait()

        def gbody(g, _):
            rows = iota + g * L
            base = j * CHUNK + g * L
            ef0 = ef0_v[pl.ds(base, L)]
            ef1 = ef1_v[pl.ds(base, L)]
            acc = mb2v
            for k in range(128 // L):
                for l in range(L):
                    col = jnp.full((L,), k * L + l, jnp.int32)
                    va = plsc.load_gather(arows_v, [rows, col])
                    vb = plsc.load_gather(brows_v, [rows, col])
                    t = va + vb + ef0 * r0[k][l] + ef1 * r1[k][l]
                    t = jnp.maximum(t, 0.0)
                    acc = acc + t * wv[k][l]
            out_v[pl.ds(base, L)] = acc
            return 0

        lax.fori_loop(0, CHUNK // L, gbody, 0)
        return 0

    lax.fori_loop(0, CPW, body, 0)
    pltpu.sync_copy(out_v, out_hbm.at[pl.ds(w * EPW, EPW)])


_edge_call = pl.kernel(
    _edge_body,
    out_type=jax.ShapeDtypeStruct((E_PAD,), jnp.float32),
    mesh=_MESH,
    compiler_params=pltpu.CompilerParams(needs_layout_passes=False),
    scratch_types=[
        pltpu.VMEM((CPW, CHUNK), jnp.int32),
        pltpu.VMEM((CPW, CHUNK), jnp.int32),
        pltpu.VMEM((EPW,), jnp.float32),
        pltpu.VMEM((EPW,), jnp.float32),
        pltpu.VMEM((4, 128), jnp.float32),
        pltpu.VMEM((CHUNK, 128), jnp.float32),
        pltpu.VMEM((CHUNK, 128), jnp.float32),
        pltpu.VMEM((EPW,), jnp.float32),
        pltpu.SemaphoreType.DMA,
        pltpu.SemaphoreType.DMA,
    ],
)


def _tc_a_body(degp_ref, x_ref, dinv_ref, xd_ref):
    deg = degp_ref[0, :] + degp_ref[1, :] + 1.0
    dinv = lax.rsqrt(deg)[:, None]
    dinv_ref[...] = dinv
    xd_ref[...] = x_ref[...] * dinv


def _tc_b_body(aggp_ref, x_ref, dinv_ref, w1_ref, b1_ref, w2_ref,
               h2_ref, td_ref):
    dinv = dinv_ref[...]
    u1 = dinv * (aggp_ref[0] + aggp_ref[1]) + (dinv * dinv) * x_ref[...]
    g1 = jnp.maximum(
        jnp.dot(u1, w1_ref[...], preferred_element_type=jnp.float32)
        + b1_ref[...], 0.0)
    h2 = jnp.dot(g1, w2_ref[...], preferred_element_type=jnp.float32)
    h2_ref[...] = h2
    td_ref[...] = h2 * dinv


def _tc_c_body(aggp_ref, h2_ref, dinv_ref, b2_ref, mw_ref, bias_ref,
               a_ref, b_ref):
    dinv = dinv_ref[...]
    g2 = (dinv * (aggp_ref[0] + aggp_ref[1])
          + (dinv * dinv) * h2_ref[...] + b2_ref[...])
    ab = jnp.dot(g2, mw_ref[...], preferred_element_type=jnp.float32) \
        + bias_ref[...]
    a_ref[...] = ab[:, :128]
    b_ref[...] = ab[:, 128:]


_TCR = 2048


def _tc_a(degp, x_p):
    return pl.pallas_call(
        _tc_a_body,
        out_shape=(jax.ShapeDtypeStruct((N_PAD, 1), jnp.float32),
                   jax.ShapeDtypeStruct((N_PAD, 128), jnp.float32)),
    )(degp, x_p)


def _tc_b(aggp, x_p, dinv, w1, b1, w2):
    nb = N_PAD // _TCR
    return pl.pallas_call(
        _tc_b_body,
        grid=(nb,),
        in_specs=[
            pl.BlockSpec((NC, _TCR, 128), lambda i: (0, i, 0)),
            pl.BlockSpec((_TCR, 128), lambda i: (i, 0)),
            pl.BlockSpec((_TCR, 1), lambda i: (i, 0)),
            pl.BlockSpec((128, 256), lambda i: (0, 0)),
            pl.BlockSpec((1, 256), lambda i: (0, 0)),
            pl.BlockSpec((256, 128), lambda i: (0, 0)),
        ],
        out_specs=(pl.BlockSpec((_TCR, 128), lambda i: (i, 0)),
                   pl.BlockSpec((_TCR, 128), lambda i: (i, 0))),
        out_shape=(jax.ShapeDtypeStruct((N_PAD, 128), jnp.float32),
                   jax.ShapeDtypeStruct((N_PAD, 128), jnp.float32)),
    )(aggp, x_p, dinv, w1, b1, w2)


def _tc_c(aggp, h2, dinv, b2, mw, bias):
    nb = N_PAD // _TCR
    return pl.pallas_call(
        _tc_c_body,
        grid=(nb,),
        in_specs=[
            pl.BlockSpec((NC, _TCR, 128), lambda i: (0, i, 0)),
            pl.BlockSpec((_TCR, 128), lambda i: (i, 0)),
            pl.BlockSpec((_TCR, 1), lambda i: (i, 0)),
            pl.BlockSpec((1, 128), lambda i: (0, 0)),
            pl.BlockSpec((128, 256), lambda i: (0, 0)),
            pl.BlockSpec((1, 256), lambda i: (0, 0)),
        ],
        out_specs=(pl.BlockSpec((_TCR, 128), lambda i: (i, 0)),
                   pl.BlockSpec((_TCR, 128), lambda i: (i, 0))),
        out_shape=(jax.ShapeDtypeStruct((N_PAD, 128), jnp.float32),
                   jax.ShapeDtypeStruct((N_PAD, 128), jnp.float32)),
    )(aggp, h2, dinv, b2, mw, bias)


def kernel(x, edge_index, edge_features, W1, b1, W2, b2, mW1, mb1, mW2, mb2):
    N, E = x.shape[0], edge_index.shape[1]
    x_p = jnp.pad(x, ((0, N_PAD - N), (0, 0)))
    src_p = jnp.pad(edge_index[0], (0, E_PAD - E))
    dst_p = jnp.pad(edge_index[1], (0, E_PAD - E), constant_values=N)
    src2d = src_p.reshape(E_PAD // CHUNK, CHUNK)
    dst2d = dst_p.reshape(E_PAD // CHUNK, CHUNK)
    ef0 = jnp.pad(edge_features[:, 0], (0, E_PAD - E))
    ef1 = jnp.pad(edge_features[:, 1], (0, E_PAD - E))

    zeros8 = jnp.zeros((8, 128), jnp.float32)
    id80 = jnp.arange(_HR, dtype=jnp.int32)
    zeros128 = jnp.zeros((RPS, 128), jnp.float32)
    consts = jnp.stack([mW1[256], mW1[257], mW2[:, 0],
                        jnp.full((128,), mb2[0], jnp.float32)])

    degout = _deg_call(dst2d, zeros8, id80)
    degp = degout.reshape(NC, N_PAD)
    dinv, xd = _tc_a(degp, x_p)

    agg1p = _segsum_call(xd, src2d, dst2d, zeros128)
    h2, td = _tc_b(agg1p, x_p, dinv, W1, b1.reshape(1, -1), W2)

    agg2p = _segsum_call(td, src2d, dst2d, zeros128)
    biasab = jnp.concatenate([mb1, jnp.zeros((128,), jnp.float32)])
    mwcat = jnp.concatenate([mW1[:128], mW1[128:256]], axis=1)
    A, B = _tc_c(agg2p, h2, dinv, b2.reshape(1, -1), mwcat,
                 biasab.reshape(1, -1))

    out = _edge_call(A, B, src2d, dst2d, ef0, ef1, consts)
    return out[:E]


# speedup vs baseline: 5.2953x; 1.5293x over previous
import functools

import jax
import jax.numpy as jnp
from jax import lax
from jax.experimental import pallas as pl
from jax.experimental.pallas import tpu as pltpu
from jax.experimental.pallas import tpu_sc as plsc

NC, NS, L = 2, 16, 16
NW = NC * NS
CHUNK = 128
CPW = 80
EPW = CPW * CHUNK
E_PAD = NW * EPW
N_PAD = 10240
RPS = N_PAD // NS

_MESH = plsc.VectorSubcoreMesh(
    core_axis_name="c", subcore_axis_name="s", num_cores=NC, num_subcores=NS)


def _wid(c, s):
    return c * NS + s


_HR = N_PAD // 128


def _deg_body(dst2d, zeros_hbm, id_hbm, out_hbm, idx_v, id_v, hist_v, acc_sh,
              sem):
    del sem
    c = lax.axis_index("c")
    s = lax.axis_index("s")
    w = _wid(c, s)
    pltpu.sync_copy(dst2d.at[pl.ds(pl.multiple_of(w * CPW, CPW), CPW)], idx_v)
    pltpu.sync_copy(id_hbm, id_v)

    @pl.when(s < _HR // 8)
    def _():
        pltpu.sync_copy(zeros_hbm, acc_sh.at[pl.ds(pl.multiple_of(s * 8, 8), 8)])
    zeros = jnp.zeros((L,), jnp.float32)
    ones = jnp.ones((L,), jnp.float32)

    def zb(f, _):
        hist_v[f // 8, pl.ds(pl.multiple_of((f % 8) * L, L), L)] = zeros
        return 0

    lax.fori_loop(0, _HR * 8, zb, 0)

    def body(f, _):
        iv = idx_v[f // 8, pl.ds(pl.multiple_of((f % 8) * L, L), L)]
        rows = lax.shift_right_logical(iv, 7)
        cols = jnp.bitwise_and(iv, 127)
        plsc.addupdate_scatter(hist_v, [rows, cols], ones)
        return 0

    lax.fori_loop(0, CPW * 8, body, 0)
    plsc.subcore_barrier()
    pltpu.sync_copy(hist_v, acc_sh.at[id_v], add=True)
    plsc.subcore_barrier()

    @pl.when(s < _HR // 8)
    def _():
        pltpu.sync_copy(acc_sh.at[pl.ds(pl.multiple_of(s * 8, 8), 8)],
                        out_hbm.at[c, pl.ds(pl.multiple_of(s * 8, 8), 8)])


_deg_call = pl.kernel(
    _deg_body,
    out_type=jax.ShapeDtypeStruct((NC, _HR, 128), jnp.float32),
    mesh=_MESH,
    compiler_params=pltpu.CompilerParams(needs_layout_passes=False),
    scratch_types=[
        pltpu.VMEM((CPW, CHUNK), jnp.int32),
        pltpu.VMEM((_HR,), jnp.int32),
        pltpu.VMEM((_HR, 128), jnp.float32),
        pltpu.VMEM_SHARED((_HR, 128), jnp.float32),
        pltpu.SemaphoreType.DMA,
    ],
)


PH = 5
CPP = CPW // PH


def _segsum_body(table, src2d, dst2d, zeros_hbm, out_hbm,
                 sidx_v, didx_v, rows0_v, rows1_v, acc_sh, sem0, sem1):
    c = lax.axis_index("c")
    s = lax.axis_index("s")
    w = _wid(c, s)
    pltpu.sync_copy(zeros_hbm, acc_sh.at[pl.ds(s * RPS, RPS)])
    plsc.subcore_barrier()

    rows = (rows0_v, rows1_v)
    sems = (sem0, sem1)

    def start(j, b):
        pltpu.async_copy(table.at[sidx_v.at[j]], rows[b], sems[b])

    def finish(j, b):
        pltpu.make_async_copy(table.at[sidx_v.at[j]], rows[b], sems[b]).wait()
        pltpu.sync_copy(rows[b], acc_sh.at[didx_v.at[j]], add=True)

    def phase(p, _):
        off = pl.multiple_of(w * CPW + p * CPP, CPP)
        pltpu.sync_copy(src2d.at[pl.ds(off, CPP)], sidx_v)
        pltpu.sync_copy(dst2d.at[pl.ds(off, CPP)], didx_v)
        start(0, 0)

        def body(jj, _):
            j0 = jj * 2
            start(j0 + 1, 1)
            finish(j0, 0)

            @pl.when(jj < CPP // 2 - 1)
            def _():
                start(j0 + 2, 0)

            finish(j0 + 1, 1)
            return 0

        lax.fori_loop(0, CPP // 2, body, 0)
        return 0

    lax.fori_loop(0, PH, phase, 0)
    plsc.subcore_barrier()
    pltpu.sync_copy(acc_sh.at[pl.ds(s * RPS, RPS)],
                    out_hbm.at[c, pl.ds(s * RPS, RPS)])


_segsum_call = pl.kernel(
    _segsum_body,
    out_type=jax.ShapeDtypeStruct((NC, N_PAD, 128), jnp.float32),
    mesh=_MESH,
    compiler_params=pltpu.CompilerParams(needs_layout_passes=False),
    scratch_types=[
        pltpu.VMEM((CPP, CHUNK), jnp.int32),
        pltpu.VMEM((CPP, CHUNK), jnp.int32),
        pltpu.VMEM((CHUNK, 128), jnp.float32),
        pltpu.VMEM((CHUNK, 128), jnp.float32),
        pltpu.VMEM_SHARED((N_PAD, 128), jnp.float32),
        pltpu.SemaphoreType.DMA,
        pltpu.SemaphoreType.DMA,
    ],
)


def _edge_body(a_hbm, b_hbm, src2d, dst2d, ef0_hbm, ef1_hbm, consts_hbm,
               out_hbm, sidx_v, didx_v, ef0_v, ef1_v, consts_v,
               ar0_v, ar1_v, br0_v, br1_v, out_v,
               sa0, sa1, sb0, sb1):
    c = lax.axis_index("c")
    s = lax.axis_index("s")
    w = _wid(c, s)
    woff = pl.multiple_of(w * CPW, CPW)
    pltpu.sync_copy(src2d.at[pl.ds(woff, CPW)], sidx_v)
    pltpu.sync_copy(dst2d.at[pl.ds(woff, CPW)], didx_v)
    pltpu.sync_copy(ef0_hbm.at[pl.ds(w * EPW, EPW)], ef0_v)
    pltpu.sync_copy(ef1_hbm.at[pl.ds(w * EPW, EPW)], ef1_v)
    pltpu.sync_copy(consts_hbm, consts_v)
    iota = lax.iota(jnp.int32, L)
    ar = (ar0_v, ar1_v)
    br = (br0_v, br1_v)
    sas = (sa0, sa1)
    sbs = (sb0, sb1)

    def start(j, b):
        pltpu.async_copy(a_hbm.at[sidx_v.at[j]], ar[b], sas[b])
        pltpu.async_copy(b_hbm.at[didx_v.at[j]], br[b], sbs[b])

    def wait(j, b):
        pltpu.make_async_copy(a_hbm.at[sidx_v.at[j]], ar[b], sas[b]).wait()
        pltpu.make_async_copy(b_hbm.at[didx_v.at[j]], br[b], sbs[b]).wait()

    def compute(j, b):
        arows_v, brows_v = ar[b], br[b]

        def gbody(g, _):
            rows = iota + g * L
            base = j * CHUNK + g * L
            ef0 = ef0_v[pl.ds(base, L)]
            ef1 = ef1_v[pl.ds(base, L)]
            accs = [consts_v[3, pl.ds(0, L)]] + [
                jnp.zeros((L,), jnp.float32)] * 3
            for k in range(128 // L):
                r0k = consts_v[0, pl.ds(k * L, L)]
                r1k = consts_v[1, pl.ds(k * L, L)]
                wk = consts_v[2, pl.ds(k * L, L)]
                for l in range(L):
                    col = jnp.full((L,), k * L + l, jnp.int32)
                    va = plsc.load_gather(arows_v, [rows, col])
                    vb = plsc.load_gather(brows_v, [rows, col])
                    t = va + vb + ef0 * r0k[l] + ef1 * r1k[l]
                    t = jnp.maximum(t, 0.0)
                    accs[l % 4] = accs[l % 4] + t * wk[l]
            out_v[pl.ds(base, L)] = (accs[0] + accs[1]) + (accs[2] + accs[3])
            return 0

        lax.fori_loop(0, CHUNK // L, gbody, 0)

    start(0, 0)

    def body(jj, _):
        j0 = jj * 2
        start(j0 + 1, 1)
        wait(j0, 0)
        compute(j0, 0)

        @pl.when(jj < CPW // 2 - 1)
        def _():
            start(j0 + 2, 0)

        wait(j0 + 1, 1)
        compute(j0 + 1, 1)
        return 0

    lax.fori_loop(0, CPW // 2, body, 0)
    pltpu.sync_copy(out_v, out_hbm.at[pl.ds(w * EPW, EPW)])


_edge_call = pl.kernel(
    _edge_body,
    out_type=jax.ShapeDtypeStruct((E_PAD,), jnp.float32),
    mesh=_MESH,
    compiler_params=pltpu.CompilerParams(needs_layout_passes=False),
    scratch_types=[
        pltpu.VMEM((CPW, CHUNK), jnp.int32),
        pltpu.VMEM((CPW, CHUNK), jnp.int32),
        pltpu.VMEM((EPW,), jnp.float32),
        pltpu.VMEM((EPW,), jnp.float32),
        pltpu.VMEM((4, 128), jnp.float32),
        pltpu.VMEM((CHUNK, 128), jnp.float32),
        pltpu.VMEM((CHUNK, 128), jnp.float32),
        pltpu.VMEM((CHUNK, 128), jnp.float32),
        pltpu.VMEM((CHUNK, 128), jnp.float32),
        pltpu.VMEM((EPW,), jnp.float32),
        pltpu.SemaphoreType.DMA,
        pltpu.SemaphoreType.DMA,
        pltpu.SemaphoreType.DMA,
        pltpu.SemaphoreType.DMA,
    ],
)


def _tc_a_body(degp_ref, x_ref, dinv_ref, xd_ref):
    deg = degp_ref[0, :] + degp_ref[1, :] + 1.0
    dinv = lax.rsqrt(deg)[:, None]
    dinv_ref[...] = dinv
    xd_ref[...] = x_ref[...] * dinv


def _tc_b_body(aggp_ref, x_ref, dinv_ref, w1_ref, b1_ref, w2_ref,
               h2_ref, td_ref):
    dinv = dinv_ref[...]
    u1 = dinv * (aggp_ref[0] + aggp_ref[1]) + (dinv * dinv) * x_ref[...]
    g1 = jnp.maximum(
        jnp.dot(u1, w1_ref[...], preferred_element_type=jnp.float32)
        + b1_ref[...], 0.0)
    h2 = jnp.dot(g1, w2_ref[...], preferred_element_type=jnp.float32)
    h2_ref[...] = h2
    td_ref[...] = h2 * dinv


def _tc_c_body(aggp_ref, h2_ref, dinv_ref, b2_ref, mw_ref, bias_ref,
               a_ref, b_ref):
    dinv = dinv_ref[...]
    g2 = (dinv * (aggp_ref[0] + aggp_ref[1])
          + (dinv * dinv) * h2_ref[...] + b2_ref[...])
    ab = jnp.dot(g2, mw_ref[...], preferred_element_type=jnp.float32) \
        + bias_ref[...]
    a_ref[...] = ab[:, :128]
    b_ref[...] = ab[:, 128:]


_TCR = 2048


def _tc_a(degp, x_p):
    return pl.pallas_call(
        _tc_a_body,
        out_shape=(jax.ShapeDtypeStruct((N_PAD, 1), jnp.float32),
                   jax.ShapeDtypeStruct((N_PAD, 128), jnp.float32)),
    )(degp, x_p)


def _tc_b(aggp, x_p, dinv, w1, b1, w2):
    nb = N_PAD // _TCR
    return pl.pallas_call(
        _tc_b_body,
        grid=(nb,),
        in_specs=[
            pl.BlockSpec((NC, _TCR, 128), lambda i: (0, i, 0)),
            pl.BlockSpec((_TCR, 128), lambda i: (i, 0)),
            pl.BlockSpec((_TCR, 1), lambda i: (i, 0)),
            pl.BlockSpec((128, 256), lambda i: (0, 0)),
            pl.BlockSpec((1, 256), lambda i: (0, 0)),
            pl.BlockSpec((256, 128), lambda i: (0, 0)),
        ],
        out_specs=(pl.BlockSpec((_TCR, 128), lambda i: (i, 0)),
                   pl.BlockSpec((_TCR, 128), lambda i: (i, 0))),
        out_shape=(jax.ShapeDtypeStruct((N_PAD, 128), jnp.float32),
                   jax.ShapeDtypeStruct((N_PAD, 128), jnp.float32)),
    )(aggp, x_p, dinv, w1, b1, w2)


def _tc_c(aggp, h2, dinv, b2, mw, bias):
    nb = N_PAD // _TCR
    return pl.pallas_call(
        _tc_c_body,
        grid=(nb,),
        in_specs=[
            pl.BlockSpec((NC, _TCR, 128), lambda i: (0, i, 0)),
            pl.BlockSpec((_TCR, 128), lambda i: (i, 0)),
            pl.BlockSpec((_TCR, 1), lambda i: (i, 0)),
            pl.BlockSpec((1, 128), lambda i: (0, 0)),
            pl.BlockSpec((128, 256), lambda i: (0, 0)),
            pl.BlockSpec((1, 256), lambda i: (0, 0)),
        ],
        out_specs=(pl.BlockSpec((_TCR, 128), lambda i: (i, 0)),
                   pl.BlockSpec((_TCR, 128), lambda i: (i, 0))),
        out_shape=(jax.ShapeDtypeStruct((N_PAD, 128), jnp.float32),
                   jax.ShapeDtypeStruct((N_PAD, 128), jnp.float32)),
    )(aggp, h2, dinv, b2, mw, bias)


def kernel(x, edge_index, edge_features, W1, b1, W2, b2, mW1, mb1, mW2, mb2):
    N, E = x.shape[0], edge_index.shape[1]
    x_p = jnp.pad(x, ((0, N_PAD - N), (0, 0)))
    src_p = jnp.pad(edge_index[0], (0, E_PAD - E))
    dst_p = jnp.pad(edge_index[1], (0, E_PAD - E), constant_values=N)
    src2d = src_p.reshape(E_PAD // CHUNK, CHUNK)
    dst2d = dst_p.reshape(E_PAD // CHUNK, CHUNK)
    ef0 = jnp.pad(edge_features[:, 0], (0, E_PAD - E))
    ef1 = jnp.pad(edge_features[:, 1], (0, E_PAD - E))

    zeros8 = jnp.zeros((8, 128), jnp.float32)
    id80 = jnp.arange(_HR, dtype=jnp.int32)
    zeros128 = jnp.zeros((RPS, 128), jnp.float32)
    consts = jnp.stack([mW1[256], mW1[257], mW2[:, 0],
                        jnp.full((128,), mb2[0], jnp.float32)])

    degout = _deg_call(dst2d, zeros8, id80)
    degp = degout.reshape(NC, N_PAD)
    dinv, xd = _tc_a(degp, x_p)

    agg1p = _segsum_call(xd, src2d, dst2d, zeros128)
    h2, td = _tc_b(agg1p, x_p, dinv, W1, b1.reshape(1, -1), W2)

    agg2p = _segsum_call(td, src2d, dst2d, zeros128)
    biasab = jnp.concatenate([mb1, jnp.zeros((128,), jnp.float32)])
    mwcat = jnp.concatenate([mW1[:128], mW1[128:256]], axis=1)
    A, B = _tc_c(agg2p, h2, dinv, b2.reshape(1, -1), mwcat,
                 biasab.reshape(1, -1))

    out = _edge_call(A, B, src2d, dst2d, ef0, ef1, consts)
    return out[:E]


# speedup vs baseline: 5.7036x; 1.0771x over previous
import functools

import jax
import jax.numpy as jnp
from jax import lax
from jax.experimental import pallas as pl
from jax.experimental.pallas import tpu as pltpu
from jax.experimental.pallas import tpu_sc as plsc

NC, NS, L = 2, 16, 16
NW = NC * NS
CHUNK = 128
CPW = 80
EPW = CPW * CHUNK
E_PAD = NW * EPW
N_PAD = 10240
RPS = N_PAD // NS

_MESH = plsc.VectorSubcoreMesh(
    core_axis_name="c", subcore_axis_name="s", num_cores=NC, num_subcores=NS)


def _wid(c, s):
    return c * NS + s


_HR = N_PAD // 128


def _deg_body(dst2d, zeros_hbm, id_hbm, out_hbm, idx_v, id_v, hist_v, acc_sh,
              sem):
    del sem
    c = lax.axis_index("c")
    s = lax.axis_index("s")
    w = _wid(c, s)
    pltpu.sync_copy(dst2d.at[pl.ds(pl.multiple_of(w * CPW, CPW), CPW)], idx_v)
    pltpu.sync_copy(id_hbm, id_v)

    @pl.when(s < _HR // 8)
    def _():
        pltpu.sync_copy(zeros_hbm, acc_sh.at[pl.ds(pl.multiple_of(s * 8, 8), 8)])
    zeros = jnp.zeros((L,), jnp.float32)
    ones = jnp.ones((L,), jnp.float32)

    def zb(f, _):
        hist_v[f // 8, pl.ds(pl.multiple_of((f % 8) * L, L), L)] = zeros
        return 0

    lax.fori_loop(0, _HR * 8, zb, 0)

    def body(f, _):
        iv = idx_v[f // 8, pl.ds(pl.multiple_of((f % 8) * L, L), L)]
        rows = lax.shift_right_logical(iv, 7)
        cols = jnp.bitwise_and(iv, 127)
        plsc.addupdate_scatter(hist_v, [rows, cols], ones)
        return 0

    lax.fori_loop(0, CPW * 8, body, 0)
    plsc.subcore_barrier()
    pltpu.sync_copy(hist_v, acc_sh.at[id_v], add=True)
    plsc.subcore_barrier()

    @pl.when(s < _HR // 8)
    def _():
        pltpu.sync_copy(acc_sh.at[pl.ds(pl.multiple_of(s * 8, 8), 8)],
                        out_hbm.at[c, pl.ds(pl.multiple_of(s * 8, 8), 8)])


_deg_call = pl.kernel(
    _deg_body,
    out_type=jax.ShapeDtypeStruct((NC, _HR, 128), jnp.float32),
    mesh=_MESH,
    compiler_params=pltpu.CompilerParams(needs_layout_passes=False),
    scratch_types=[
        pltpu.VMEM((CPW, CHUNK), jnp.int32),
        pltpu.VMEM((_HR,), jnp.int32),
        pltpu.VMEM((_HR, 128), jnp.float32),
        pltpu.VMEM_SHARED((_HR, 128), jnp.float32),
        pltpu.SemaphoreType.DMA,
    ],
)


PH = 5
CPP = CPW // PH


def _segsum_body(table, src2d, dst2d, zeros_hbm, out_hbm,
                 sidx_v, didx_v, rows0_v, rows1_v, acc_sh, sem0, sem1):
    c = lax.axis_index("c")
    s = lax.axis_index("s")
    w = _wid(c, s)
    pltpu.sync_copy(zeros_hbm, acc_sh.at[pl.ds(s * RPS, RPS)])
    plsc.subcore_barrier()

    rows = (rows0_v, rows1_v)
    sems = (sem0, sem1)

    def start(j, b):
        pltpu.async_copy(table.at[sidx_v.at[j]], rows[b], sems[b])

    def finish(j, b):
        pltpu.make_async_copy(table.at[sidx_v.at[j]], rows[b], sems[b]).wait()
        pltpu.sync_copy(rows[b], acc_sh.at[didx_v.at[j]], add=True)

    def phase(p, _):
        off = pl.multiple_of(w * CPW + p * CPP, CPP)
        pltpu.sync_copy(src2d.at[pl.ds(off, CPP)], sidx_v)
        pltpu.sync_copy(dst2d.at[pl.ds(off, CPP)], didx_v)
        start(0, 0)

        def body(jj, _):
            j0 = jj * 2
            start(j0 + 1, 1)
            finish(j0, 0)

            @pl.when(jj < CPP // 2 - 1)
            def _():
                start(j0 + 2, 0)

            finish(j0 + 1, 1)
            return 0

        lax.fori_loop(0, CPP // 2, body, 0)
        return 0

    lax.fori_loop(0, PH, phase, 0)
    plsc.subcore_barrier()
    pltpu.sync_copy(acc_sh.at[pl.ds(s * RPS, RPS)],
                    out_hbm.at[c, pl.ds(s * RPS, RPS)])


_segsum_call = pl.kernel(
    _segsum_body,
    out_type=jax.ShapeDtypeStruct((NC, N_PAD, 128), jnp.float32),
    mesh=_MESH,
    compiler_params=pltpu.CompilerParams(needs_layout_passes=False),
    scratch_types=[
        pltpu.VMEM((CPP, CHUNK), jnp.int32),
        pltpu.VMEM((CPP, CHUNK), jnp.int32),
        pltpu.VMEM((CHUNK, 128), jnp.float32),
        pltpu.VMEM((CHUNK, 128), jnp.float32),
        pltpu.VMEM_SHARED((N_PAD, 128), jnp.float32),
        pltpu.SemaphoreType.DMA,
        pltpu.SemaphoreType.DMA,
    ],
)


EC = 32
ECPW = EPW // EC
PH_E = 20
CPP_E = ECPW // PH_E
EPP = CPP_E * EC


def _edge_body(ab32, src2d, dst2d, ef0_hbm, ef1_hbm, cp_hbm, mbt_hbm,
               out_hbm, sidx_v, didx_v, ef0_v, ef1_v, cp_v, mbt_v,
               ar0_v, ar1_v, br0_v, br1_v, out_v, ab_sh,
               sa0, sa1, sb0, sb1):
    c = lax.axis_index("c")
    s = lax.axis_index("s")
    w = _wid(c, s)
    soff = pl.multiple_of(s * RPS, 8)
    pltpu.sync_copy(ab32.at[pl.ds(soff, RPS)], ab_sh.at[pl.ds(soff, RPS)])
    pltpu.sync_copy(cp_hbm, cp_v)
    pltpu.sync_copy(mbt_hbm, mbt_v)
    plsc.subcore_barrier()
    iota = lax.iota(jnp.int32, L)
    mb2v = mbt_v[0, pl.ds(0, L)]
    zero_f = jnp.zeros((L,), jnp.float32)
    bzero = jnp.zeros((2 * L,), jnp.bfloat16)
    ar = (ar0_v, ar1_v)
    br = (br0_v, br1_v)
    sas = (sa0, sa1)
    sbs = (sb0, sb1)

    def start(jl, b):
        pltpu.async_copy(ab_sh.at[sidx_v.at[jl]], ar[b], sas[b])
        pltpu.async_copy(ab_sh.at[didx_v.at[jl]], br[b], sbs[b])

    def wait(jl, b):
        pltpu.make_async_copy(ab_sh.at[sidx_v.at[jl]], ar[b], sas[b]).wait()
        pltpu.make_async_copy(ab_sh.at[didx_v.at[jl]], br[b], sbs[b]).wait()

    def compute(jl, b):
        arows_v, brows_v = ar[b], br[b]
        base0 = jl * EC
        efp = []
        for g in range(EC // L):
            e0 = ef0_v[pl.ds(base0 + g * L, L)]
            e1 = ef1_v[pl.ds(base0 + g * L, L)]
            efp.append(
                (plsc.pack(e0, e0, format=plsc.PackFormat.INTERLEAVED),
                 plsc.pack(e1, e1, format=plsc.PackFormat.INTERLEAVED)))

        def cbody(c2, accs):
            r0p = plsc.bitcast(cp_v[0, c2], jnp.bfloat16)
            r1p = plsc.bitcast(cp_v[1, c2], jnp.bfloat16)
            wp = plsc.bitcast(cp_v[2, c2], jnp.bfloat16)
            cola = jnp.full((L,), c2, jnp.int32)
            colb = cola + 64
            nxt = []
            for g in range(EC // L):
                rows = iota + g * L
                va = plsc.bitcast(
                    plsc.load_gather(arows_v, [rows, cola]), jnp.bfloat16)
                vb = plsc.bitcast(
                    plsc.load_gather(brows_v, [rows, colb]), jnp.bfloat16)
                t = va + vb + efp[g][0] * r0p + efp[g][1] * r1p
                t = jnp.maximum(t, bzero)
                t0, t1 = plsc.unpack(t * wp,
                                     format=plsc.PackFormat.INTERLEAVED)
                nxt.append((accs[g][0] + t0, accs[g][1] + t1))
            return tuple(nxt)

        init = tuple((zero_f, zero_f) for _ in range(EC // L))
        accs = lax.fori_loop(0, 64, cbody, init)
        for g in range(EC // L):
            out_v[pl.ds(base0 + g * L, L)] = accs[g][0] + accs[g][1] + mb2v

    def phase(p, _):
        off = pl.multiple_of(w * ECPW + p * CPP_E, 8)
        pltpu.sync_copy(src2d.at[pl.ds(off, CPP_E)], sidx_v)
        pltpu.sync_copy(dst2d.at[pl.ds(off, CPP_E)], didx_v)
        eoff = pl.multiple_of(w * EPW + p * EPP, 8)
        pltpu.sync_copy(ef0_hbm.at[pl.ds(eoff, EPP)], ef0_v)
        pltpu.sync_copy(ef1_hbm.at[pl.ds(eoff, EPP)], ef1_v)
        start(0, 0)

        def body(jj, _):
            j0 = jj * 2
            start(j0 + 1, 1)
            wait(j0, 0)
            compute(j0, 0)

            @pl.when(jj < CPP_E // 2 - 1)
            def _():
                start(j0 + 2, 0)

            wait(j0 + 1, 1)
            compute(j0 + 1, 1)
            return 0

        lax.fori_loop(0, CPP_E // 2, body, 0)
        pltpu.sync_copy(out_v, out_hbm.at[pl.ds(eoff, EPP)])
        return 0

    lax.fori_loop(0, PH_E, phase, 0)


_edge_call = pl.kernel(
    _edge_body,
    out_type=jax.ShapeDtypeStruct((E_PAD,), jnp.float32),
    mesh=_MESH,
    compiler_params=pltpu.CompilerParams(needs_layout_passes=False),
    scratch_types=[
        pltpu.VMEM((CPP_E, EC), jnp.int32),
        pltpu.VMEM((CPP_E, EC), jnp.int32),
        pltpu.VMEM((EPP,), jnp.float32),
        pltpu.VMEM((EPP,), jnp.float32),
        pltpu.VMEM((3, 64, L), jnp.int32),
        pltpu.VMEM((8, L), jnp.float32),
        pltpu.VMEM((EC, 128), jnp.int32),
        pltpu.VMEM((EC, 128), jnp.int32),
        pltpu.VMEM((EC, 128), jnp.int32),
        pltpu.VMEM((EC, 128), jnp.int32),
        pltpu.VMEM((EPP,), jnp.float32),
        pltpu.VMEM_SHARED((N_PAD, 128), jnp.int32),
        pltpu.SemaphoreType.DMA,
        pltpu.SemaphoreType.DMA,
        pltpu.SemaphoreType.DMA,
        pltpu.SemaphoreType.DMA,
    ],
)


def _tc_a_body(degp_ref, x_ref, dinv_ref, xd_ref):
    deg = degp_ref[0, :] + degp_ref[1, :] + 1.0
    dinv = lax.rsqrt(deg)[:, None]
    dinv_ref[...] = dinv
    xd_ref[...] = x_ref[...] * dinv


def _tc_b_body(aggp_ref, x_ref, dinv_ref, w1_ref, b1_ref, w2_ref,
               h2_ref, td_ref):
    dinv = dinv_ref[...]
    u1 = dinv * (aggp_ref[0] + aggp_ref[1]) + (dinv * dinv) * x_ref[...]
    g1 = jnp.maximum(
        jnp.dot(u1, w1_ref[...], preferred_element_type=jnp.float32)
        + b1_ref[...], 0.0)
    h2 = jnp.dot(g1, w2_ref[...], preferred_element_type=jnp.float32)
    h2_ref[...] = h2
    td_ref[...] = h2 * dinv


def _tc_c_body(aggp_ref, h2_ref, dinv_ref, b2_ref, mw_ref, bias_ref,
               a_ref, b_ref):
    dinv = dinv_ref[...]
    g2 = (dinv * (aggp_ref[0] + aggp_ref[1])
          + (dinv * dinv) * h2_ref[...] + b2_ref[...])
    ab = jnp.dot(g2, mw_ref[...], preferred_element_type=jnp.float32) \
        + bias_ref[...]
    a_ref[...] = ab[:, :128].astype(jnp.bfloat16)
    b_ref[...] = ab[:, 128:].astype(jnp.bfloat16)


_TCR = 2048


def _tc_a(degp, x_p):
    return pl.pallas_call(
        _tc_a_body,
        out_shape=(jax.ShapeDtypeStruct((N_PAD, 1), jnp.float32),
                   jax.ShapeDtypeStruct((N_PAD, 128), jnp.float32)),
    )(degp, x_p)


def _tc_b(aggp, x_p, dinv, w1, b1, w2):
    nb = N_PAD // _TCR
    return pl.pallas_call(
        _tc_b_body,
        grid=(nb,),
        in_specs=[
            pl.BlockSpec((NC, _TCR, 128), lambda i: (0, i, 0)),
            pl.BlockSpec((_TCR, 128), lambda i: (i, 0)),
            pl.BlockSpec((_TCR, 1), lambda i: (i, 0)),
            pl.BlockSpec((128, 256), lambda i: (0, 0)),
            pl.BlockSpec((1, 256), lambda i: (0, 0)),
            pl.BlockSpec((256, 128), lambda i: (0, 0)),
        ],
        out_specs=(pl.BlockSpec((_TCR, 128), lambda i: (i, 0)),
                   pl.BlockSpec((_TCR, 128), lambda i: (i, 0))),
        out_shape=(jax.ShapeDtypeStruct((N_PAD, 128), jnp.float32),
                   jax.ShapeDtypeStruct((N_PAD, 128), jnp.float32)),
    )(aggp, x_p, dinv, w1, b1, w2)


def _tc_c(aggp, h2, dinv, b2, mw, bias):
    nb = N_PAD // _TCR
    return pl.pallas_call(
        _tc_c_body,
        grid=(nb,),
        in_specs=[
            pl.BlockSpec((NC, _TCR, 128), lambda i: (0, i, 0)),
            pl.BlockSpec((_TCR, 128), lambda i: (i, 0)),
            pl.BlockSpec((_TCR, 1), lambda i: (i, 0)),
            pl.BlockSpec((1, 128), lambda i: (0, 0)),
            pl.BlockSpec((128, 256), lambda i: (0, 0)),
            pl.BlockSpec((1, 256), lambda i: (0, 0)),
        ],
        out_specs=(pl.BlockSpec((_TCR, 128), lambda i: (i, 0)),
                   pl.BlockSpec((_TCR, 128), lambda i: (i, 0))),
        out_shape=(jax.ShapeDtypeStruct((N_PAD, 128), jnp.bfloat16),
                   jax.ShapeDtypeStruct((N_PAD, 128), jnp.bfloat16)),
    )(aggp, h2, dinv, b2, mw, bias)


def kernel(x, edge_index, edge_features, W1, b1, W2, b2, mW1, mb1, mW2, mb2):
    N, E = x.shape[0], edge_index.shape[1]
    x_p = jnp.pad(x, ((0, N_PAD - N), (0, 0)))
    src_p = jnp.pad(edge_index[0], (0, E_PAD - E))
    dst_p = jnp.pad(edge_index[1], (0, E_PAD - E), constant_values=N)
    src2d = src_p.reshape(E_PAD // CHUNK, CHUNK)
    dst2d = dst_p.reshape(E_PAD // CHUNK, CHUNK)
    ef0 = jnp.pad(edge_features[:, 0], (0, E_PAD - E))
    ef1 = jnp.pad(edge_features[:, 1], (0, E_PAD - E))

    zeros8 = jnp.zeros((8, 128), jnp.float32)
    id80 = jnp.arange(_HR, dtype=jnp.int32)
    zeros128 = jnp.zeros((RPS, 128), jnp.float32)

    def packpairs(v):
        vb = v.astype(jnp.bfloat16).reshape(64, 2)
        return lax.bitcast_convert_type(vb, jnp.int32)

    cp = jnp.stack([jnp.tile(packpairs(mW1[256])[:, None], (1, L)),
                    jnp.tile(packpairs(mW1[257])[:, None], (1, L)),
                    jnp.tile(packpairs(mW2[:, 0])[:, None], (1, L))])
    mbt = jnp.full((8, L), mb2[0], jnp.float32)

    degout = _deg_call(dst2d, zeros8, id80)
    degp = degout.reshape(NC, N_PAD)
    dinv, xd = _tc_a(degp, x_p)

    agg1p = _segsum_call(xd, src2d, dst2d, zeros128)
    h2, td = _tc_b(agg1p, x_p, dinv, W1, b1.reshape(1, -1), W2)

    agg2p = _segsum_call(td, src2d, dst2d, zeros128)
    biasab = jnp.concatenate([mb1, jnp.zeros((128,), jnp.float32)])
    mwcat = jnp.concatenate([mW1[:128], mW1[128:256]], axis=1)
    A, B = _tc_c(agg2p, h2, dinv, b2.reshape(1, -1), mwcat,
                 biasab.reshape(1, -1))

    a32 = lax.bitcast_convert_type(A.reshape(N_PAD, 64, 2), jnp.int32)
    b32 = lax.bitcast_convert_type(B.reshape(N_PAD, 64, 2), jnp.int32)
    ab32 = jnp.concatenate([a32, b32], axis=1)
    src2de = src_p.reshape(E_PAD // EC, EC)
    dst2de = dst_p.reshape(E_PAD // EC, EC)
    out = _edge_call(ab32, src2de, dst2de, ef0, ef1, cp, mbt)
    return out[:E]
